# baseline (speedup 1.0000x reference)
import functools

import jax
import jax.numpy as jnp
from jax import lax
from jax.experimental import pallas as pl
from jax.experimental.pallas import tpu as pltpu
from jax.experimental.pallas import tpu_sc as plsc

N = 10000
E = 320000
D = 128
DH = D // 2
ED = 16
CW = 8
NC, NS = 2, 16
EPT = E // NS
K = 160
NCHUNK = EPT // K
NCB = -(-NCHUNK // 2)

CZ = 80
NROWCHUNK = N // CZ
NZ = -(-NROWCHUNK // NS)

_mesh = plsc.VectorSubcoreMesh(
    core_axis_name="c", subcore_axis_name="s", num_cores=NC, num_subcores=NS)


@functools.partial(
    pl.kernel,
    out_type=(jax.ShapeDtypeStruct((NC * N, DH), jnp.float32),
              jax.ShapeDtypeStruct((NC * N, CW), jnp.float32)),
    mesh=_mesh,
    compiler_params=pltpu.CompilerParams(use_tc_tiling_on_sc=False),
    scratch_types=[
        pltpu.VMEM((NCHUNK, K), jnp.int32),
        pltpu.VMEM((NCHUNK, K), jnp.int32),
        pltpu.VMEM((3, K, DH), jnp.float32),
        pltpu.VMEM((K, CW), jnp.float32),
        pltpu.VMEM((CZ, DH), jnp.float32),
        pltpu.VMEM((CZ, CW), jnp.float32),
        pltpu.VMEM_SHARED((N, DH), jnp.float32),
        pltpu.VMEM_SHARED((N, CW), jnp.float32),
        pltpu.SemaphoreType.DMA,
        pltpu.SemaphoreType.DMA,
        pltpu.SemaphoreType.DMA,
        pltpu.SemaphoreType.DMA,
    ],
)
def _sc_xcount(xl_hbm, xr_hbm, src_hbm, dst_hbm, ones_hbm, zx_hbm, zc_hbm,
               outx_hbm, outc_hbm,
               srcb, dstb, rows, onesv, stx, stc, accx, accc,
               isem, gsem, ssem, csem):
    cid = lax.axis_index("c")
    sid = lax.axis_index("s")

    pltpu.sync_copy(zx_hbm, stx)
    pltpu.sync_copy(zc_hbm, stc)
    pltpu.sync_copy(ones_hbm, onesv)

    @pl.loop(0, NZ)
    def _(j):
        idx = j * NS + sid

        @pl.when(idx < NROWCHUNK)
        def _():
            r0 = idx * CZ
            pltpu.sync_copy(stx, accx.at[pl.ds(r0, CZ)])
            pltpu.sync_copy(stc, accc.at[pl.ds(r0, CZ)])

    plsc.subcore_barrier()

    base0 = sid * EPT

    @pl.loop(0, NCHUNK)
    def _(j):
        pltpu.async_copy(
            src_hbm.at[pl.ds(base0 + j * K, K)], srcb.at[j], isem)
        pltpu.async_copy(
            dst_hbm.at[pl.ds(base0 + j * K, K)], dstb.at[j], isem)

    @pl.loop(0, NCHUNK)
    def _(j):
        pltpu.make_async_copy(
            src_hbm.at[pl.ds(base0 + j * K, K)], srcb.at[j], isem).wait()
        pltpu.make_async_copy(
            dst_hbm.at[pl.ds(base0 + j * K, K)], dstb.at[j], isem).wait()

    def issue_gather(c, b):
        @pl.when(cid == 0)
        def _():
            pltpu.async_copy(xl_hbm.at[srcb.at[c]], rows.at[b], gsem)

        @pl.when(cid == 1)
        def _():
            pltpu.async_copy(xr_hbm.at[srcb.at[c]], rows.at[b], gsem)

    def wait_gather(c, b):
        pltpu.make_async_copy(xl_hbm.at[srcb.at[c]], rows.at[b], gsem).wait()

    def issue_scatter(c, b):
        pltpu.async_copy(rows.at[b], accx.at[dstb.at[c]], ssem, add=True)

    def wait_scatter(c, b):
        pltpu.make_async_copy(rows.at[b], accx.at[dstb.at[c]], ssem).wait()

    def issue_cnt_scatter(c):
        pltpu.async_copy(onesv, accc.at[dstb.at[c]], csem, add=True)

    def wait_cnt_scatter(c):
        pltpu.make_async_copy(onesv, accc.at[dstb.at[c]], csem).wait()

    issue_gather(0, 0)

    @pl.loop(0, NCHUNK)
    def _(c):
        b = lax.rem(c, 3)
        wait_gather(c, b)

        @pl.when(c >= 2)
        def _():
            wait_scatter(c - 2, lax.rem(c - 2, 3))

        @pl.when(c + 1 < NCHUNK)
        def _():
            issue_gather(c + 1, lax.rem(c + 1, 3))

        issue_scatter(c, b)

        @pl.when(lax.rem(c, 2) == cid)
        def _():
            @pl.when(c >= 2)
            def _():
                wait_cnt_scatter(c - 2)

            issue_cnt_scatter(c)

    wait_scatter(NCHUNK - 2, lax.rem(NCHUNK - 2, 3))
    wait_scatter(NCHUNK - 1, lax.rem(NCHUNK - 1, 3))
    c_last = 2 * lax.div(NCHUNK - 1 - cid, 2) + cid
    wait_cnt_scatter(c_last)

    plsc.subcore_barrier()

    @pl.loop(0, NZ)
    def _(j):
        idx = j * NS + sid

        @pl.when(idx < NROWCHUNK)
        def _():
            r0 = idx * CZ
            pltpu.sync_copy(accx.at[pl.ds(r0, CZ)], stx)
            pltpu.sync_copy(stx, outx_hbm.at[pl.ds(cid * N + r0, CZ)])
            pltpu.sync_copy(accc.at[pl.ds(r0, CZ)], stc)
            pltpu.sync_copy(stc, outc_hbm.at[pl.ds(cid * N + r0, CZ)])


@functools.partial(
    pl.kernel,
    out_type=jax.ShapeDtypeStruct((NC * N, ED), jnp.float32),
    mesh=_mesh,
    compiler_params=pltpu.CompilerParams(use_tc_tiling_on_sc=False),
    scratch_types=[
        pltpu.VMEM((NCB, K), jnp.int32),
        pltpu.VMEM((2, K, ED), jnp.float32),
        pltpu.VMEM((CZ, ED), jnp.float32),
        pltpu.VMEM_SHARED((N, ED), jnp.float32),
        pltpu.SemaphoreType.DMA,
        pltpu.SemaphoreType.DMA,
        pltpu.SemaphoreType.DMA,
    ],
)
def _sc_edge(ea_hbm, dst_hbm, ze_hbm, oute_hbm,
             dstb, eav, ste, acce, isem, elsem, essem):
    cid = lax.axis_index("c")
    sid = lax.axis_index("s")

    pltpu.sync_copy(ze_hbm, ste)

    @pl.loop(0, NZ)
    def _(j):
        idx = j * NS + sid

        @pl.when(idx < NROWCHUNK)
        def _():
            pltpu.sync_copy(ste, acce.at[pl.ds(idx * CZ, CZ)])

    plsc.subcore_barrier()

    base0 = sid * EPT

    def chunk_of(j):
        return 2 * j + cid

    @pl.loop(0, NCB)
    def _(j):
        c = chunk_of(j)

        @pl.when(c < NCHUNK)
        def _():
            pltpu.async_copy(
                dst_hbm.at[pl.ds(base0 + c * K, K)], dstb.at[j], isem)

    @pl.loop(0, NCB)
    def _(j):
        c = chunk_of(j)

        @pl.when(c < NCHUNK)
        def _():
            pltpu.make_async_copy(
                dst_hbm.at[pl.ds(base0 + c * K, K)], dstb.at[j], isem).wait()

    def issue_ea_load(j, b):
        base = base0 + chunk_of(j) * K
        pltpu.async_copy(ea_hbm.at[pl.ds(base, K)], eav.at[b], elsem)

    def wait_ea_load(j, b):
        base = base0 + chunk_of(j) * K
        pltpu.make_async_copy(
            ea_hbm.at[pl.ds(base, K)], eav.at[b], elsem).wait()

    def issue_ea_scatter(j, b):
        pltpu.async_copy(eav.at[b], acce.at[dstb.at[j]], essem, add=True)

    def wait_ea_scatter(j, b):
        pltpu.make_async_copy(eav.at[b], acce.at[dstb.at[j]], essem).wait()

    nact = NCB - jnp.where(cid == 1, NCHUNK % 2, 0)

    issue_ea_load(0, 0)

    @pl.loop(0, NCB)
    def _(j):
        @pl.when(chunk_of(j) < NCHUNK)
        def _():
            b = lax.rem(j, 2)
            wait_ea_load(j, b)

            @pl.when(j >= 1)
            def _():
                wait_ea_scatter(j - 1, 1 - b)

            @pl.when(chunk_of(j + 1) < NCHUNK)
            def _():
                issue_ea_load(j + 1, 1 - b)

            issue_ea_scatter(j, b)

    wait_ea_scatter(nact - 1, lax.rem(nact - 1, 2))

    plsc.subcore_barrier()

    @pl.loop(0, NZ)
    def _(j):
        idx = j * NS + sid

        @pl.when(idx < NROWCHUNK)
        def _():
            r0 = idx * CZ
            pltpu.sync_copy(acce.at[pl.ds(r0, CZ)], ste)
            pltpu.sync_copy(ste, oute_hbm.at[pl.ds(cid * N + r0, CZ)])


BN = 2000


def _tc_body(px_ref, pe_ref, pc_ref, x_ref, we_ref, wl_ref, wr_ref,
             be_ref, bl_ref, br_ref, g_ref, b_ref, o_ref):
    sx = jnp.concatenate([px_ref[0], px_ref[1]], axis=1)
    se = pe_ref[0] + pe_ref[1]
    cnt = (pc_ref[0] + pc_ref[1])[:, :1]
    num = sx + jnp.dot(se, we_ref[...], preferred_element_type=jnp.float32)
    num = num + cnt * be_ref[...]
    agg = num / jnp.maximum(cnt, 1.0)
    out = (jnp.dot(agg, wl_ref[...], preferred_element_type=jnp.float32)
           + jnp.dot(x_ref[...], wr_ref[...], preferred_element_type=jnp.float32)
           + bl_ref[...] + br_ref[...])
    mu = jnp.mean(out, axis=1, keepdims=True)
    ctr = out - mu
    var = jnp.mean(ctr * ctr, axis=1, keepdims=True)
    o_ref[...] = ctr * lax.rsqrt(var + 1e-5) * g_ref[...] + b_ref[...]


_tc_combine = pl.pallas_call(
    _tc_body,
    grid=(N // BN,),
    in_specs=[
        pl.BlockSpec((NC, BN, DH), lambda i: (0, i, 0)),
        pl.BlockSpec((NC, BN, ED), lambda i: (0, i, 0)),
        pl.BlockSpec((NC, BN, CW), lambda i: (0, i, 0)),
        pl.BlockSpec((BN, D), lambda i: (i, 0)),
        pl.BlockSpec((ED, D), lambda i: (0, 0)),
        pl.BlockSpec((D, D), lambda i: (0, 0)),
        pl.BlockSpec((D, D), lambda i: (0, 0)),
        pl.BlockSpec((1, D), lambda i: (0, 0)),
        pl.BlockSpec((1, D), lambda i: (0, 0)),
        pl.BlockSpec((1, D), lambda i: (0, 0)),
        pl.BlockSpec((1, D), lambda i: (0, 0)),
        pl.BlockSpec((1, D), lambda i: (0, 0)),
    ],
    out_specs=pl.BlockSpec((BN, D), lambda i: (i, 0)),
    out_shape=jax.ShapeDtypeStruct((N, D), jnp.float32),
)


def kernel(x, edge_index, edge_attr, W_edge, b_edge, W_l, b_l, W_r, b_r,
           gamma, beta):
    src = edge_index[0].astype(jnp.int32)
    dst = edge_index[1].astype(jnp.int32)
    xl = x[:, :DH]
    xr = x[:, DH:]
    ones = jnp.ones((K, CW), jnp.float32)
    zx = jnp.zeros((CZ, DH), jnp.float32)
    ze = jnp.zeros((CZ, ED), jnp.float32)
    zc = jnp.zeros((CZ, CW), jnp.float32)
    px, pc = _sc_xcount(xl, xr, src, dst, ones, zx, zc)
    pe = _sc_edge(edge_attr, dst, ze)
    px = px.reshape(NC, N, DH)
    pe = pe.reshape(NC, N, ED)
    pc = pc.reshape(NC, N, CW)
    return _tc_combine(px, pe, pc, x, W_edge, W_l, W_r,
                       b_edge.reshape(1, D), b_l.reshape(1, D),
                       b_r.reshape(1, D), gamma.reshape(1, D),
                       beta.reshape(1, D))


# speedup vs baseline: 1.0461x; 1.0461x over previous
import functools

import jax
import jax.numpy as jnp
from jax import lax
from jax.experimental import pallas as pl
from jax.experimental.pallas import tpu as pltpu
from jax.experimental.pallas import tpu_sc as plsc

N = 10000
E = 320000
D = 128
DH = D // 2
ED = 16
CW = 8
NC, NS = 2, 16
EPT = E // NS
K = 200
NCHUNK = EPT // K
KB = 160
NCHUNKB = EPT // KB
NCB = -(-NCHUNKB // 2)

CZ = 80
NROWCHUNK = N // CZ
NZ = -(-NROWCHUNK // NS)

_mesh = plsc.VectorSubcoreMesh(
    core_axis_name="c", subcore_axis_name="s", num_cores=NC, num_subcores=NS)


@functools.partial(
    pl.kernel,
    out_type=(jax.ShapeDtypeStruct((NC * N, DH), jnp.float32),
              jax.ShapeDtypeStruct((NC * N, CW), jnp.float32)),
    mesh=_mesh,
    compiler_params=pltpu.CompilerParams(use_tc_tiling_on_sc=False),
    scratch_types=[
        pltpu.VMEM((NCHUNK, K), jnp.int32),
        pltpu.VMEM((NCHUNK, K), jnp.int32),
        pltpu.VMEM((3, K, DH), jnp.float32),
        pltpu.VMEM((K, CW), jnp.float32),
        pltpu.VMEM((CZ, DH), jnp.float32),
        pltpu.VMEM((CZ, CW), jnp.float32),
        pltpu.VMEM_SHARED((N, DH), jnp.float32),
        pltpu.VMEM_SHARED((N, CW), jnp.float32),
        pltpu.SemaphoreType.DMA,
        pltpu.SemaphoreType.DMA,
        pltpu.SemaphoreType.DMA,
        pltpu.SemaphoreType.DMA,
    ],
)
def _sc_xcount(xl_hbm, xr_hbm, src_hbm, dst_hbm, ones_hbm, zx_hbm, zc_hbm,
               outx_hbm, outc_hbm,
               srcb, dstb, rows, onesv, stx, stc, accx, accc,
               isem, gsem, ssem, csem):
    cid = lax.axis_index("c")
    sid = lax.axis_index("s")

    pltpu.sync_copy(zx_hbm, stx)
    pltpu.sync_copy(zc_hbm, stc)
    pltpu.sync_copy(ones_hbm, onesv)

    @pl.loop(0, NZ)
    def _(j):
        idx = j * NS + sid

        @pl.when(idx < NROWCHUNK)
        def _():
            r0 = idx * CZ
            pltpu.sync_copy(stx, accx.at[pl.ds(r0, CZ)])
            pltpu.sync_copy(stc, accc.at[pl.ds(r0, CZ)])

    plsc.subcore_barrier()

    base0 = sid * EPT

    @pl.loop(0, NCHUNK)
    def _(j):
        pltpu.async_copy(
            src_hbm.at[pl.ds(base0 + j * K, K)], srcb.at[j], isem)
        pltpu.async_copy(
            dst_hbm.at[pl.ds(base0 + j * K, K)], dstb.at[j], isem)

    @pl.loop(0, NCHUNK)
    def _(j):
        pltpu.make_async_copy(
            src_hbm.at[pl.ds(base0 + j * K, K)], srcb.at[j], isem).wait()
        pltpu.make_async_copy(
            dst_hbm.at[pl.ds(base0 + j * K, K)], dstb.at[j], isem).wait()

    def issue_gather(c, b):
        @pl.when(cid == 0)
        def _():
            pltpu.async_copy(xl_hbm.at[srcb.at[c]], rows.at[b], gsem)

        @pl.when(cid == 1)
        def _():
            pltpu.async_copy(xr_hbm.at[srcb.at[c]], rows.at[b], gsem)

    def wait_gather(c, b):
        pltpu.make_async_copy(xl_hbm.at[srcb.at[c]], rows.at[b], gsem).wait()

    def issue_scatter(c, b):
        pltpu.async_copy(rows.at[b], accx.at[dstb.at[c]], ssem, add=True)

    def wait_scatter(c, b):
        pltpu.make_async_copy(rows.at[b], accx.at[dstb.at[c]], ssem).wait()

    def issue_cnt_scatter(c):
        pltpu.async_copy(onesv, accc.at[dstb.at[c]], csem, add=True)

    def wait_cnt_scatter(c):
        pltpu.make_async_copy(onesv, accc.at[dstb.at[c]], csem).wait()

    issue_gather(0, 0)

    @pl.loop(0, NCHUNK)
    def _(c):
        b = lax.rem(c, 3)
        wait_gather(c, b)

        @pl.when(c >= 2)
        def _():
            wait_scatter(c - 2, lax.rem(c - 2, 3))

        @pl.when(c + 1 < NCHUNK)
        def _():
            issue_gather(c + 1, lax.rem(c + 1, 3))

        issue_scatter(c, b)

        @pl.when(lax.rem(c, 2) == cid)
        def _():
            @pl.when(c >= 2)
            def _():
                wait_cnt_scatter(c - 2)

            issue_cnt_scatter(c)

    wait_scatter(NCHUNK - 2, lax.rem(NCHUNK - 2, 3))
    wait_scatter(NCHUNK - 1, lax.rem(NCHUNK - 1, 3))
    c_last = 2 * lax.div(NCHUNK - 1 - cid, 2) + cid
    wait_cnt_scatter(c_last)

    plsc.subcore_barrier()

    @pl.loop(0, NZ)
    def _(j):
        idx = j * NS + sid

        @pl.when(idx < NROWCHUNK)
        def _():
            r0 = idx * CZ
            pltpu.sync_copy(accx.at[pl.ds(r0, CZ)], stx)
            pltpu.sync_copy(stx, outx_hbm.at[pl.ds(cid * N + r0, CZ)])
            pltpu.sync_copy(accc.at[pl.ds(r0, CZ)], stc)
            pltpu.sync_copy(stc, outc_hbm.at[pl.ds(cid * N + r0, CZ)])


@functools.partial(
    pl.kernel,
    out_type=jax.ShapeDtypeStruct((NC * N, ED), jnp.float32),
    mesh=_mesh,
    compiler_params=pltpu.CompilerParams(use_tc_tiling_on_sc=False),
    scratch_types=[
        pltpu.VMEM((NCB, KB), jnp.int32),
        pltpu.VMEM((2, KB, ED), jnp.float32),
        pltpu.VMEM((CZ, ED), jnp.float32),
        pltpu.VMEM_SHARED((N, ED), jnp.float32),
        pltpu.SemaphoreType.DMA,
        pltpu.SemaphoreType.DMA,
        pltpu.SemaphoreType.DMA,
    ],
)
def _sc_edge(ea_hbm, dst_hbm, ze_hbm, oute_hbm,
             dstb, eav, ste, acce, isem, elsem, essem):
    cid = lax.axis_index("c")
    sid = lax.axis_index("s")

    pltpu.sync_copy(ze_hbm, ste)

    @pl.loop(0, NZ)
    def _(j):
        idx = j * NS + sid

        @pl.when(idx < NROWCHUNK)
        def _():
            pltpu.sync_copy(ste, acce.at[pl.ds(idx * CZ, CZ)])

    plsc.subcore_barrier()

    base0 = sid * EPT

    def chunk_of(j):
        return 2 * j + cid

    @pl.loop(0, NCB)
    def _(j):
        c = chunk_of(j)

        @pl.when(c < NCHUNKB)
        def _():
            pltpu.async_copy(
                dst_hbm.at[pl.ds(base0 + c * KB, KB)], dstb.at[j], isem)

    @pl.loop(0, NCB)
    def _(j):
        c = chunk_of(j)

        @pl.when(c < NCHUNKB)
        def _():
            pltpu.make_async_copy(
                dst_hbm.at[pl.ds(base0 + c * KB, KB)], dstb.at[j], isem).wait()

    def issue_ea_load(j, b):
        base = base0 + chunk_of(j) * KB
        pltpu.async_copy(ea_hbm.at[pl.ds(base, KB)], eav.at[b], elsem)

    def wait_ea_load(j, b):
        base = base0 + chunk_of(j) * KB
        pltpu.make_async_copy(
            ea_hbm.at[pl.ds(base, KB)], eav.at[b], elsem).wait()

    def issue_ea_scatter(j, b):
        pltpu.async_copy(eav.at[b], acce.at[dstb.at[j]], essem, add=True)

    def wait_ea_scatter(j, b):
        pltpu.make_async_copy(eav.at[b], acce.at[dstb.at[j]], essem).wait()

    nact = NCB - jnp.where(cid == 1, NCHUNKB % 2, 0)

    issue_ea_load(0, 0)

    @pl.loop(0, NCB)
    def _(j):
        @pl.when(chunk_of(j) < NCHUNKB)
        def _():
            b = lax.rem(j, 2)
            wait_ea_load(j, b)

            @pl.when(j >= 1)
            def _():
                wait_ea_scatter(j - 1, 1 - b)

            @pl.when(chunk_of(j + 1) < NCHUNKB)
            def _():
                issue_ea_load(j + 1, 1 - b)

            issue_ea_scatter(j, b)

    wait_ea_scatter(nact - 1, lax.rem(nact - 1, 2))

    plsc.subcore_barrier()

    @pl.loop(0, NZ)
    def _(j):
        idx = j * NS + sid

        @pl.when(idx < NROWCHUNK)
        def _():
            r0 = idx * CZ
            pltpu.sync_copy(acce.at[pl.ds(r0, CZ)], ste)
            pltpu.sync_copy(ste, oute_hbm.at[pl.ds(cid * N + r0, CZ)])


BN = 2000


def _tc_body(px_ref, pe_ref, pc_ref, x_ref, we_ref, wl_ref, wr_ref,
             be_ref, bl_ref, br_ref, g_ref, b_ref, o_ref):
    sx = jnp.concatenate([px_ref[0], px_ref[1]], axis=1)
    se = pe_ref[0] + pe_ref[1]
    cnt = (pc_ref[0] + pc_ref[1])[:, :1]
    num = sx + jnp.dot(se, we_ref[...], preferred_element_type=jnp.float32)
    num = num + cnt * be_ref[...]
    agg = num / jnp.maximum(cnt, 1.0)
    out = (jnp.dot(agg, wl_ref[...], preferred_element_type=jnp.float32)
           + jnp.dot(x_ref[...], wr_ref[...], preferred_element_type=jnp.float32)
           + bl_ref[...] + br_ref[...])
    mu = jnp.mean(out, axis=1, keepdims=True)
    ctr = out - mu
    var = jnp.mean(ctr * ctr, axis=1, keepdims=True)
    o_ref[...] = ctr * lax.rsqrt(var + 1e-5) * g_ref[...] + b_ref[...]


_tc_combine = pl.pallas_call(
    _tc_body,
    grid=(N // BN,),
    in_specs=[
        pl.BlockSpec((NC, BN, DH), lambda i: (0, i, 0)),
        pl.BlockSpec((NC, BN, ED), lambda i: (0, i, 0)),
        pl.BlockSpec((NC, BN, CW), lambda i: (0, i, 0)),
        pl.BlockSpec((BN, D), lambda i: (i, 0)),
        pl.BlockSpec((ED, D), lambda i: (0, 0)),
        pl.BlockSpec((D, D), lambda i: (0, 0)),
        pl.BlockSpec((D, D), lambda i: (0, 0)),
        pl.BlockSpec((1, D), lambda i: (0, 0)),
        pl.BlockSpec((1, D), lambda i: (0, 0)),
        pl.BlockSpec((1, D), lambda i: (0, 0)),
        pl.BlockSpec((1, D), lambda i: (0, 0)),
        pl.BlockSpec((1, D), lambda i: (0, 0)),
    ],
    out_specs=pl.BlockSpec((BN, D), lambda i: (i, 0)),
    out_shape=jax.ShapeDtypeStruct((N, D), jnp.float32),
)


def kernel(x, edge_index, edge_attr, W_edge, b_edge, W_l, b_l, W_r, b_r,
           gamma, beta):
    src = edge_index[0].astype(jnp.int32)
    dst = edge_index[1].astype(jnp.int32)
    xl = x[:, :DH]
    xr = x[:, DH:]
    ones = jnp.ones((K, CW), jnp.float32)
    zx = jnp.zeros((CZ, DH), jnp.float32)
    ze = jnp.zeros((CZ, ED), jnp.float32)
    zc = jnp.zeros((CZ, CW), jnp.float32)
    px, pc = _sc_xcount(xl, xr, src, dst, ones, zx, zc)
    pe = _sc_edge(edge_attr, dst, ze)
    px = px.reshape(NC, N, DH)
    pe = pe.reshape(NC, N, ED)
    pc = pc.reshape(NC, N, CW)
    return _tc_combine(px, pe, pc, x, W_edge, W_l, W_r,
                       b_edge.reshape(1, D), b_l.reshape(1, D),
                       b_r.reshape(1, D), gamma.reshape(1, D),
                       beta.reshape(1, D))


# speedup vs baseline: 1.0702x; 1.0231x over previous
import functools

import jax
import jax.numpy as jnp
from jax import lax
from jax.experimental import pallas as pl
from jax.experimental.pallas import tpu as pltpu
from jax.experimental.pallas import tpu_sc as plsc

N = 10000
E = 320000
D = 128
DH = D // 2
ED = 16
CW = 8
NC, NS = 2, 16
EPT = E // NS
K = 200
NCHUNK = EPT // K
KB = 200
NCHUNKB = EPT // KB
NCB = -(-NCHUNKB // 2)

CZ = 80
NROWCHUNK = N // CZ
NZ = -(-NROWCHUNK // NS)

_mesh = plsc.VectorSubcoreMesh(
    core_axis_name="c", subcore_axis_name="s", num_cores=NC, num_subcores=NS)


@functools.partial(
    pl.kernel,
    out_type=(jax.ShapeDtypeStruct((NC * N, DH), jnp.float32),
              jax.ShapeDtypeStruct((NC * N, CW), jnp.float32)),
    mesh=_mesh,
    compiler_params=pltpu.CompilerParams(use_tc_tiling_on_sc=False),
    scratch_types=[
        pltpu.VMEM((NCHUNK, K), jnp.int32),
        pltpu.VMEM((NCHUNK, K), jnp.int32),
        pltpu.VMEM((3, K, DH), jnp.float32),
        pltpu.VMEM((K, CW), jnp.float32),
        pltpu.VMEM((CZ, DH), jnp.float32),
        pltpu.VMEM((CZ, CW), jnp.float32),
        pltpu.VMEM_SHARED((N, DH), jnp.float32),
        pltpu.VMEM_SHARED((N, CW), jnp.float32),
        pltpu.SemaphoreType.DMA,
        pltpu.SemaphoreType.DMA,
        pltpu.SemaphoreType.DMA,
        pltpu.SemaphoreType.DMA,
    ],
)
def _sc_xcount(xl_hbm, xr_hbm, src_hbm, dst_hbm, ones_hbm, zx_hbm, zc_hbm,
               outx_hbm, outc_hbm,
               srcb, dstb, rows, onesv, stx, stc, accx, accc,
               isem, gsem, ssem, csem):
    cid = lax.axis_index("c")
    sid = lax.axis_index("s")

    pltpu.sync_copy(zx_hbm, stx)
    pltpu.sync_copy(zc_hbm, stc)
    pltpu.sync_copy(ones_hbm, onesv)

    @pl.loop(0, NZ)
    def _(j):
        idx = j * NS + sid

        @pl.when(idx < NROWCHUNK)
        def _():
            r0 = idx * CZ
            pltpu.sync_copy(stx, accx.at[pl.ds(r0, CZ)])
            pltpu.sync_copy(stc, accc.at[pl.ds(r0, CZ)])

    plsc.subcore_barrier()

    base0 = sid * EPT

    @pl.loop(0, NCHUNK)
    def _(j):
        pltpu.async_copy(
            src_hbm.at[pl.ds(base0 + j * K, K)], srcb.at[j], isem)
        pltpu.async_copy(
            dst_hbm.at[pl.ds(base0 + j * K, K)], dstb.at[j], isem)

    @pl.loop(0, NCHUNK)
    def _(j):
        pltpu.make_async_copy(
            src_hbm.at[pl.ds(base0 + j * K, K)], srcb.at[j], isem).wait()
        pltpu.make_async_copy(
            dst_hbm.at[pl.ds(base0 + j * K, K)], dstb.at[j], isem).wait()

    def issue_gather(c, b):
        @pl.when(cid == 0)
        def _():
            pltpu.async_copy(xl_hbm.at[srcb.at[c]], rows.at[b], gsem)

        @pl.when(cid == 1)
        def _():
            pltpu.async_copy(xr_hbm.at[srcb.at[c]], rows.at[b], gsem)

    def wait_gather(c, b):
        pltpu.make_async_copy(xl_hbm.at[srcb.at[c]], rows.at[b], gsem).wait()

    def issue_scatter(c, b):
        pltpu.async_copy(rows.at[b], accx.at[dstb.at[c]], ssem, add=True)

    def wait_scatter(c, b):
        pltpu.make_async_copy(rows.at[b], accx.at[dstb.at[c]], ssem).wait()

    def issue_cnt_scatter(c):
        pltpu.async_copy(onesv, accc.at[dstb.at[c]], csem, add=True)

    def wait_cnt_scatter(c):
        pltpu.make_async_copy(onesv, accc.at[dstb.at[c]], csem).wait()

    issue_gather(0, 0)

    @pl.loop(0, NCHUNK)
    def _(c):
        b = lax.rem(c, 3)
        wait_gather(c, b)

        @pl.when(c >= 2)
        def _():
            wait_scatter(c - 2, lax.rem(c - 2, 3))

        @pl.when(c + 1 < NCHUNK)
        def _():
            issue_gather(c + 1, lax.rem(c + 1, 3))

        issue_scatter(c, b)

        @pl.when(lax.rem(c, 2) == cid)
        def _():
            @pl.when(c >= 2)
            def _():
                wait_cnt_scatter(c - 2)

            issue_cnt_scatter(c)

    wait_scatter(NCHUNK - 2, lax.rem(NCHUNK - 2, 3))
    wait_scatter(NCHUNK - 1, lax.rem(NCHUNK - 1, 3))
    c_last = 2 * lax.div(NCHUNK - 1 - cid, 2) + cid
    wait_cnt_scatter(c_last)

    plsc.subcore_barrier()

    @pl.loop(0, NZ)
    def _(j):
        idx = j * NS + sid

        @pl.when(idx < NROWCHUNK)
        def _():
            r0 = idx * CZ
            pltpu.sync_copy(accx.at[pl.ds(r0, CZ)], stx)
            pltpu.sync_copy(stx, outx_hbm.at[pl.ds(cid * N + r0, CZ)])
            pltpu.sync_copy(accc.at[pl.ds(r0, CZ)], stc)
            pltpu.sync_copy(stc, outc_hbm.at[pl.ds(cid * N + r0, CZ)])


@functools.partial(
    pl.kernel,
    out_type=jax.ShapeDtypeStruct((NC * N, ED), jnp.float32),
    mesh=_mesh,
    compiler_params=pltpu.CompilerParams(use_tc_tiling_on_sc=False),
    scratch_types=[
        pltpu.VMEM((NCB, KB), jnp.int32),
        pltpu.VMEM((2, KB, ED), jnp.float32),
        pltpu.VMEM((CZ, ED), jnp.float32),
        pltpu.VMEM_SHARED((N, ED), jnp.float32),
        pltpu.SemaphoreType.DMA,
        pltpu.SemaphoreType.DMA,
        pltpu.SemaphoreType.DMA,
    ],
)
def _sc_edge(ea_hbm, dst_hbm, ze_hbm, oute_hbm,
             dstb, eav, ste, acce, isem, elsem, essem):
    cid = lax.axis_index("c")
    sid = lax.axis_index("s")

    pltpu.sync_copy(ze_hbm, ste)

    @pl.loop(0, NZ)
    def _(j):
        idx = j * NS + sid

        @pl.when(idx < NROWCHUNK)
        def _():
            pltpu.sync_copy(ste, acce.at[pl.ds(idx * CZ, CZ)])

    plsc.subcore_barrier()

    base0 = sid * EPT

    def chunk_of(j):
        return 2 * j + cid

    @pl.loop(0, NCB)
    def _(j):
        c = chunk_of(j)

        @pl.when(c < NCHUNKB)
        def _():
            pltpu.async_copy(
                dst_hbm.at[pl.ds(base0 + c * KB, KB)], dstb.at[j], isem)

    @pl.loop(0, NCB)
    def _(j):
        c = chunk_of(j)

        @pl.when(c < NCHUNKB)
        def _():
            pltpu.make_async_copy(
                dst_hbm.at[pl.ds(base0 + c * KB, KB)], dstb.at[j], isem).wait()

    def issue_ea_load(j, b):
        base = base0 + chunk_of(j) * KB
        pltpu.async_copy(ea_hbm.at[pl.ds(base, KB)], eav.at[b], elsem)

    def wait_ea_load(j, b):
        base = base0 + chunk_of(j) * KB
        pltpu.make_async_copy(
            ea_hbm.at[pl.ds(base, KB)], eav.at[b], elsem).wait()

    def issue_ea_scatter(j, b):
        pltpu.async_copy(eav.at[b], acce.at[dstb.at[j]], essem, add=True)

    def wait_ea_scatter(j, b):
        pltpu.make_async_copy(eav.at[b], acce.at[dstb.at[j]], essem).wait()

    nact = NCB - jnp.where(cid == 1, NCHUNKB % 2, 0)

    issue_ea_load(0, 0)

    @pl.loop(0, NCB)
    def _(j):
        @pl.when(chunk_of(j) < NCHUNKB)
        def _():
            b = lax.rem(j, 2)
            wait_ea_load(j, b)

            @pl.when(j >= 1)
            def _():
                wait_ea_scatter(j - 1, 1 - b)

            @pl.when(chunk_of(j + 1) < NCHUNKB)
            def _():
                issue_ea_load(j + 1, 1 - b)

            issue_ea_scatter(j, b)

    wait_ea_scatter(nact - 1, lax.rem(nact - 1, 2))

    plsc.subcore_barrier()

    @pl.loop(0, NZ)
    def _(j):
        idx = j * NS + sid

        @pl.when(idx < NROWCHUNK)
        def _():
            r0 = idx * CZ
            pltpu.sync_copy(acce.at[pl.ds(r0, CZ)], ste)
            pltpu.sync_copy(ste, oute_hbm.at[pl.ds(cid * N + r0, CZ)])


BN = 2000


def _tc_body(px_ref, pe_ref, pc_ref, x_ref, we_ref, wl_ref, wr_ref,
             be_ref, bl_ref, br_ref, g_ref, b_ref, o_ref):
    sx = jnp.concatenate([px_ref[0], px_ref[1]], axis=1)
    se = pe_ref[0] + pe_ref[1]
    cnt = (pc_ref[0] + pc_ref[1])[:, :1]
    num = sx + jnp.dot(se, we_ref[...], preferred_element_type=jnp.float32)
    num = num + cnt * be_ref[...]
    agg = num / jnp.maximum(cnt, 1.0)
    out = (jnp.dot(agg, wl_ref[...], preferred_element_type=jnp.float32)
           + jnp.dot(x_ref[...], wr_ref[...], preferred_element_type=jnp.float32)
           + bl_ref[...] + br_ref[...])
    mu = jnp.mean(out, axis=1, keepdims=True)
    ctr = out - mu
    var = jnp.mean(ctr * ctr, axis=1, keepdims=True)
    o_ref[...] = ctr * lax.rsqrt(var + 1e-5) * g_ref[...] + b_ref[...]


_tc_combine = pl.pallas_call(
    _tc_body,
    grid=(N // BN,),
    in_specs=[
        pl.BlockSpec((NC, BN, DH), lambda i: (0, i, 0)),
        pl.BlockSpec((NC, BN, ED), lambda i: (0, i, 0)),
        pl.BlockSpec((NC, BN, CW), lambda i: (0, i, 0)),
        pl.BlockSpec((BN, D), lambda i: (i, 0)),
        pl.BlockSpec((ED, D), lambda i: (0, 0)),
        pl.BlockSpec((D, D), lambda i: (0, 0)),
        pl.BlockSpec((D, D), lambda i: (0, 0)),
        pl.BlockSpec((1, D), lambda i: (0, 0)),
        pl.BlockSpec((1, D), lambda i: (0, 0)),
        pl.BlockSpec((1, D), lambda i: (0, 0)),
        pl.BlockSpec((1, D), lambda i: (0, 0)),
        pl.BlockSpec((1, D), lambda i: (0, 0)),
    ],
    out_specs=pl.BlockSpec((BN, D), lambda i: (i, 0)),
    out_shape=jax.ShapeDtypeStruct((N, D), jnp.float32),
)


def kernel(x, edge_index, edge_attr, W_edge, b_edge, W_l, b_l, W_r, b_r,
           gamma, beta):
    src = edge_index[0].astype(jnp.int32)
    dst = edge_index[1].astype(jnp.int32)
    xl = x[:, :DH]
    xr = x[:, DH:]
    ones = jnp.ones((K, CW), jnp.float32)
    zx = jnp.zeros((CZ, DH), jnp.float32)
    ze = jnp.zeros((CZ, ED), jnp.float32)
    zc = jnp.zeros((CZ, CW), jnp.float32)
    px, pc = _sc_xcount(xl, xr, src, dst, ones, zx, zc)
    pe = _sc_edge(edge_attr, dst, ze)
    px = px.reshape(NC, N, DH)
    pe = pe.reshape(NC, N, ED)
    pc = pc.reshape(NC, N, CW)
    return _tc_combine(px, pe, pc, x, W_edge, W_l, W_r,
                       b_edge.reshape(1, D), b_l.reshape(1, D),
                       b_r.reshape(1, D), gamma.reshape(1, D),
                       beta.reshape(1, D))
